# baseline (speedup 1.0000x reference)
import functools

import jax
import jax.numpy as jnp
from jax import lax
from jax.experimental import pallas as pl
from jax.experimental.pallas import tpu as pltpu
from jax.experimental.pallas import tpu_sc as plsc

_L = 16
_CH = 128


@functools.lru_cache(maxsize=None)
def _build_mf(B, D, NU, NI):
    info = plsc.get_sparse_core_info()
    NC, NS = info.num_cores, info.num_subcores
    NW = NC * NS
    assert B % (NW * _CH) == 0 and D % _L == 0
    bw = B // NW
    nch = bw // _CH
    mesh = plsc.VectorSubcoreMesh(core_axis_name="c", subcore_axis_name="s")

    @functools.partial(
        pl.kernel,
        out_type=(
            jax.ShapeDtypeStruct((B, D), jnp.float32),
            jax.ShapeDtypeStruct((B, D), jnp.float32),
            jax.ShapeDtypeStruct((B, D), jnp.float32),
            jax.ShapeDtypeStruct((NW, _L), jnp.float32),
        ),
        scratch_types=(
            pltpu.VMEM((nch, _CH), jnp.int32),
            pltpu.VMEM((nch, _CH), jnp.int32),
            pltpu.VMEM((nch, _CH), jnp.int32),
            pltpu.VMEM((bw, D), jnp.float32),
            pltpu.VMEM((bw, D), jnp.float32),
            pltpu.VMEM((bw, D), jnp.float32),
            pltpu.VMEM((_L,), jnp.float32),
            pltpu.SemaphoreType.DMA,
            pltpu.SemaphoreType.DMA,
            pltpu.SemaphoreType.DMA,
        ),
        mesh=mesh,
        compiler_params=pltpu.CompilerParams(use_tc_tiling_on_sc=False),
    )
    def mf(user_idx, pos_idx, neg_idx, user_table, item_table,
           user_out, pos_out, neg_out, part_out,
           idx_u, idx_p, idx_n, rows_u, rows_p, rows_n, acc_v,
           sem_u, sem_p, sem_n):
        wid = lax.axis_index("s") * NC + lax.axis_index("c")
        base = wid * bw

        pltpu.sync_copy(user_idx.at[pl.ds(wid * nch, nch)], idx_u)
        pltpu.sync_copy(pos_idx.at[pl.ds(wid * nch, nch)], idx_p)
        pltpu.sync_copy(neg_idx.at[pl.ds(wid * nch, nch)], idx_n)

        cps = []
        for tbl, idx, rows, sem in (
            (user_table, idx_u, rows_u, sem_u),
            (item_table, idx_p, rows_p, sem_p),
            (item_table, idx_n, rows_n, sem_n),
        ):
            cps.append([
                pltpu.async_copy(tbl.at[idx.at[j]],
                                 rows.at[pl.ds(j * _CH, _CH)], sem)
                for j in range(nch)
            ])

        def sumsq(rows, acc):
            def body(r, a):
                for c in range(D // _L):
                    v = rows[r, pl.ds(c * _L, _L)]
                    a = a + v * v
                return a
            return lax.fori_loop(0, bw, body, acc)

        acc = jnp.zeros((_L,), jnp.float32)
        for k, (rows, out) in enumerate(
                ((rows_u, user_out), (rows_p, pos_out), (rows_n, neg_out))):
            for c in cps[k]:
                c.wait()
            acc = sumsq(rows, acc)
            pltpu.sync_copy(rows, out.at[pl.ds(base, bw)])

        acc_v[...] = acc
        pltpu.sync_copy(acc_v, part_out.at[wid])

    return mf


def kernel(user_list, pos_items, neg_items, user_table, item_table):
    B = user_list.shape[0]
    D = user_table.shape[1]
    mf = _build_mf(B, D, user_table.shape[0], item_table.shape[0])
    u2 = user_list.reshape(-1, _CH)
    p2 = pos_items.reshape(-1, _CH)
    n2 = neg_items.reshape(-1, _CH)
    user_emb, pos_emb, neg_emb, parts = mf(u2, p2, n2, user_table, item_table)
    reg = jnp.sum(parts) / B
    return (user_emb, pos_emb, neg_emb, reg)


# speedup vs baseline: 1.0052x; 1.0052x over previous
import functools

import jax
import jax.numpy as jnp
from jax import lax
from jax.experimental import pallas as pl
from jax.experimental.pallas import tpu as pltpu
from jax.experimental.pallas import tpu_sc as plsc

_L = 16
_CH = 128


@functools.lru_cache(maxsize=None)
def _build_gather(B, D, NV):
    info = plsc.get_sparse_core_info()
    NC, NS = info.num_cores, info.num_subcores
    NW = NC * NS
    assert B % (NW * _CH) == 0 and D % _L == 0
    bw = B // NW
    nch = bw // _CH
    mesh = plsc.VectorSubcoreMesh(core_axis_name="c", subcore_axis_name="s")

    @functools.partial(
        pl.kernel,
        out_type=(
            jax.ShapeDtypeStruct((B, D), jnp.float32),
            jax.ShapeDtypeStruct((NW, _L), jnp.float32),
        ),
        scratch_types=(
            pltpu.VMEM((nch, _CH), jnp.int32),
            pltpu.VMEM((bw, D), jnp.float32),
            pltpu.SemaphoreType.DMA,
        ),
        mesh=mesh,
        compiler_params=pltpu.CompilerParams(use_tc_tiling_on_sc=False),
    )
    def emb(idx_hbm, table, out, part_out, idx_v, rows_v, sem):
        wid = lax.axis_index("s") * NC + lax.axis_index("c")
        base = wid * bw

        pltpu.sync_copy(idx_hbm.at[pl.ds(wid * nch, nch)], idx_v)
        cps = [
            pltpu.async_copy(table.at[idx_v.at[j]],
                             rows_v.at[pl.ds(j * _CH, _CH)], sem)
            for j in range(nch)
        ]

        def chunk_sumsq(j, acc):
            def body(r, a):
                for c in range(D // _L):
                    v = rows_v[r, pl.ds(c * _L, _L)]
                    a = a + v * v
                return a
            return lax.fori_loop(j * _CH, (j + 1) * _CH, body, acc)

        acc = jnp.zeros((_L,), jnp.float32)
        for j in range(nch):
            cps[j].wait()
            acc = chunk_sumsq(j, acc)
        pltpu.sync_copy(rows_v, out.at[pl.ds(base, bw)])

        rows_v[0, pl.ds(0, _L)] = acc
        pltpu.sync_copy(rows_v.at[0, pl.ds(0, _L)], part_out.at[wid])

    return emb


def kernel(user_list, pos_items, neg_items, user_table, item_table):
    B = user_list.shape[0]
    D = user_table.shape[1]
    emb_u = _build_gather(B, D, user_table.shape[0])
    emb_i = _build_gather(B, D, item_table.shape[0])
    u2 = user_list.reshape(-1, _CH)
    p2 = pos_items.reshape(-1, _CH)
    n2 = neg_items.reshape(-1, _CH)
    user_emb, part_u = emb_u(u2, user_table)
    pos_emb, part_p = emb_i(p2, item_table)
    neg_emb, part_n = emb_i(n2, item_table)
    reg = (jnp.sum(part_u) + jnp.sum(part_p) + jnp.sum(part_n)) / B
    return (user_emb, pos_emb, neg_emb, reg)


# speedup vs baseline: 3.3131x; 3.2960x over previous
import functools

import jax
import jax.numpy as jnp
from jax import lax
from jax.experimental import pallas as pl
from jax.experimental.pallas import tpu as pltpu
from jax.experimental.pallas import tpu_sc as plsc

_L = 16
_TPB = 128
_JW = 256
_NBUF = 4
_KG = 2
_JBLK = 8


@functools.lru_cache(maxsize=None)
def _build_scan(B, D, NV):
    info = plsc.get_sparse_core_info()
    NC, NS = info.num_cores, info.num_subcores
    NW = NC * NS
    JMAX = (NV + _TPB - 1) // _TPB
    J = NW * _JW
    assert J >= JMAX and D % _L == 0
    B2 = 2 * B
    mesh = plsc.VectorSubcoreMesh(core_axis_name="c", subcore_axis_name="s")

    @functools.partial(
        pl.kernel,
        out_type=(
            jax.ShapeDtypeStruct((B * D,), jnp.float32),
            jax.ShapeDtypeStruct((B2 * D,), jnp.float32),
            jax.ShapeDtypeStruct((NW, _L), jnp.float32),
        ),
        scratch_types=(
            pltpu.VMEM((B + _L,), jnp.int32),
            pltpu.VMEM((B2 + _L,), jnp.int32),
            pltpu.VMEM((_JW + _L,), jnp.int32),
            pltpu.VMEM((_JW + _L,), jnp.int32),
            pltpu.VMEM((_JW + _L,), jnp.int32),
            pltpu.VMEM((_JW + _L,), jnp.int32),
            pltpu.VMEM((_NBUF, D, _TPB), jnp.float32),
            pltpu.VMEM((_KG, D), jnp.float32),
            pltpu.VMEM((_L,), jnp.float32),
            pltpu.SemaphoreType.DMA,
            pltpu.SemaphoreType.DMA,
            pltpu.SemaphoreType.DMA,
            pltpu.SemaphoreType.DMA,
            pltpu.SemaphoreType.DMA,
            pltpu.SemaphoreType.DMA,
        ),
        mesh=mesh,
        compiler_params=pltpu.CompilerParams(needs_layout_passes=False),
    )
    def mf(utab_t, itab_t, rec_u, off_u, cnt_u, rec_i, off_i, cnt_i,
           out_u, out_i, part_out,
           recv_u, recv_i, offu_v, cntu_v, offi_v, cnti_v,
           bufs, colstage, accst,
           semt0, semt1, semt2, semt3, semo0, semo1):
        semt = (semt0, semt1, semt2, semt3)
        semo = (semo0, semo1)
        wid = lax.axis_index("s") * NC + lax.axis_index("c")
        j0 = wid * _JW

        pltpu.sync_copy(rec_u, recv_u.at[pl.ds(0, B)])
        pltpu.sync_copy(rec_i, recv_i.at[pl.ds(0, B2)])
        pltpu.sync_copy(off_u.at[pl.ds(j0, _JW)], offu_v.at[pl.ds(0, _JW)])
        pltpu.sync_copy(cnt_u.at[pl.ds(j0, _JW)], cntu_v.at[pl.ds(0, _JW)])
        pltpu.sync_copy(off_i.at[pl.ds(j0, _JW)], offi_v.at[pl.ds(0, _JW)])
        pltpu.sync_copy(cnt_i.at[pl.ds(j0, _JW)], cnti_v.at[pl.ds(0, _JW)])

        dvecs = [lax.iota(jnp.int32, _L) + c * _L for c in range(D // _L)]

        def scan(tab, recv, offv, cntv, out_flat, carry):
            def fire(jloc, slot):
                jg = j0 + jloc
                @pl.when((jloc < _JW) & (jg < JMAX))
                def _():
                    o = pl.multiple_of(jg * _TPB, _TPB)
                    pltpu.async_copy(tab.at[:, pl.ds(o, _TPB)],
                                     bufs.at[slot], semt[slot])

            def wait_slot(jloc, slot):
                jg = j0 + jloc
                @pl.when(jg < JMAX)
                def _():
                    pltpu.make_async_copy(tab.at[:, pl.ds(0, _TPB)],
                                          bufs.at[slot], semt[slot]).wait()

            for s in range(_NBUF):
                fire(s, s)

            def jw_body(jw, carry):
                offs = offv[pl.ds(jw * _JBLK, _L)]
                cnts = cntv[pl.ds(jw * _JBLK, _L)]
                for jl in range(_JBLK):
                    jloc = jw * _JBLK + jl
                    slot = jl % _NBUF
                    svec = jnp.full((_L,), slot, jnp.int32)
                    wait_slot(jloc, slot)
                    o0 = offs[jl]
                    n = cnts[jl]

                    def grp_body(g, c2, o0=o0, n=n, svec=svec):
                        acc2, f0, f1 = c2
                        fs = [f0, f1]
                        for k in range(_KG):
                            h = g * _KG + k
                            vmaski = ((h - n) >> 31) & 1
                            hs = h * vmaski + (n - 1) * (1 - vmaski)
                            rv = recv[pl.ds(o0 + hs, _L)]
                            rec = rv[0]
                            b = rec >> 7
                            u = rec & (_TPB - 1)
                            usplat = jnp.full((_L,), 0, jnp.int32) + u
                            vmaskf = vmaski.astype(jnp.float32)
                            vs = []
                            for c in range(D // _L):
                                v = plsc.load_gather(
                                    bufs, [svec, dvecs[c], usplat])
                                vs.append(v)
                                acc2 = acc2 + (v * v) * vmaskf

                            @pl.when(vmaski > 0)
                            def _(k=k, b=b, vs=vs, fk=fs[k]):
                                @pl.when(fk >= 1)
                                def _():
                                    pltpu.make_async_copy(
                                        out_flat.at[pl.ds(0, D)],
                                        colstage.at[k], semo[k]).wait()
                                for c in range(D // _L):
                                    colstage[k, pl.ds(c * _L, _L)] = vs[c]
                                dst = pl.multiple_of(b * D, 8)
                                pltpu.async_copy(colstage.at[k],
                                                 out_flat.at[pl.ds(dst, D)],
                                                 semo[k])

                            fs[k] = fs[k] + vmaski
                        return (acc2, fs[0], fs[1])

                    ng = (n + (_KG - 1)) >> 1
                    carry = lax.fori_loop(0, ng, grp_body, carry)
                    fire(jloc + _NBUF, slot)
                return carry

            return lax.fori_loop(0, _JW // _JBLK, jw_body, carry)

        carry = (jnp.zeros((_L,), jnp.float32), jnp.int32(0), jnp.int32(0))
        carry = scan(utab_t, recv_u, offu_v, cntu_v, out_u, carry)
        carry = scan(itab_t, recv_i, offi_v, cnti_v, out_i, carry)
        acc, f0, f1 = carry

        for k, fk in enumerate((f0, f1)):
            @pl.when(fk >= 1)
            def _(k=k):
                pltpu.make_async_copy(out_u.at[pl.ds(0, D)],
                                      colstage.at[k], semo[k]).wait()

        accst[...] = acc
        pltpu.sync_copy(accst, part_out.at[wid])

    return mf


def _prep(idx, J):
    n = idx.shape[0]
    j = idx >> 7
    skey = (j << 16) | jnp.arange(n, dtype=jnp.int32)
    s = jnp.sort(skey)
    order = s & ((1 << 16) - 1)
    rec = (order << 7) | (jnp.take(idx, order) & (_TPB - 1))
    cnt = jnp.bincount(j, length=J).astype(jnp.int32)
    off = (jnp.cumsum(cnt) - cnt).astype(jnp.int32)
    return rec.astype(jnp.int32), off, cnt


def kernel(user_list, pos_items, neg_items, user_table, item_table):
    B = user_list.shape[0]
    D = user_table.shape[1]
    NV = user_table.shape[0]
    mf = _build_scan(B, D, NV)
    info = plsc.get_sparse_core_info()
    J = info.num_cores * info.num_subcores * _JW
    rec_u, off_u, cnt_u = _prep(user_list, J)
    idx_i = jnp.concatenate([pos_items, neg_items])
    rec_i, off_i, cnt_i = _prep(idx_i, J)
    out_u, out_i, parts = mf(user_table.T, item_table.T,
                             rec_u, off_u, cnt_u, rec_i, off_i, cnt_i)
    user_emb = out_u.reshape(B, D)
    pn = out_i.reshape(2 * B, D)
    reg = jnp.sum(parts) / B
    return (user_emb, pn[:B], pn[B:], reg)


# speedup vs baseline: 3.3234x; 1.0031x over previous
import functools

import jax
import jax.numpy as jnp
from jax import lax
from jax.experimental import pallas as pl
from jax.experimental.pallas import tpu as pltpu
from jax.experimental.pallas import tpu_sc as plsc

_L = 16
_TPB = 128
_JW = 256
_NBUF = 4
_KG = 2
_JBLK = 8


@functools.lru_cache(maxsize=None)
def _build_scan(NB, D, NV):
    info = plsc.get_sparse_core_info()
    NC, NS = info.num_cores, info.num_subcores
    NW = NC * NS
    JMAX = (NV + _TPB - 1) // _TPB
    J = NW * _JW
    assert J >= JMAX and D % _L == 0
    mesh = plsc.VectorSubcoreMesh(core_axis_name="c", subcore_axis_name="s")

    @functools.partial(
        pl.kernel,
        out_type=(
            jax.ShapeDtypeStruct((NB * D,), jnp.float32),
            jax.ShapeDtypeStruct((NW, _L), jnp.float32),
        ),
        scratch_types=(
            pltpu.VMEM((NB + _L,), jnp.int32),
            pltpu.VMEM((_JW + _L,), jnp.int32),
            pltpu.VMEM((_JW + _L,), jnp.int32),
            pltpu.VMEM((_NBUF, D, _TPB), jnp.float32),
            pltpu.VMEM((_KG, D), jnp.float32),
            pltpu.VMEM((_L,), jnp.float32),
            pltpu.SemaphoreType.DMA,
            pltpu.SemaphoreType.DMA,
            pltpu.SemaphoreType.DMA,
            pltpu.SemaphoreType.DMA,
            pltpu.SemaphoreType.DMA,
            pltpu.SemaphoreType.DMA,
        ),
        mesh=mesh,
        compiler_params=pltpu.CompilerParams(needs_layout_passes=False),
    )
    def mf(tab, rec, off, cnt,
           out_flat, part_out,
           recv, offv, cntv, bufs, colstage, accst,
           semt0, semt1, semt2, semt3, semo0, semo1):
        semt = (semt0, semt1, semt2, semt3)
        semo = (semo0, semo1)
        wid = lax.axis_index("s") * NC + lax.axis_index("c")
        j0 = wid * _JW

        pltpu.sync_copy(rec, recv.at[pl.ds(0, NB)])
        pltpu.sync_copy(off.at[pl.ds(j0, _JW)], offv.at[pl.ds(0, _JW)])
        pltpu.sync_copy(cnt.at[pl.ds(j0, _JW)], cntv.at[pl.ds(0, _JW)])

        dvecs = [lax.iota(jnp.int32, _L) + c * _L for c in range(D // _L)]

        def fire(jloc, slot):
            jg = j0 + jloc
            @pl.when((jloc < _JW) & (jg < JMAX))
            def _():
                o = pl.multiple_of(jg * _TPB, _TPB)
                pltpu.async_copy(tab.at[:, pl.ds(o, _TPB)],
                                 bufs.at[slot], semt[slot])

        def wait_slot(jloc, slot):
            jg = j0 + jloc
            @pl.when(jg < JMAX)
            def _():
                pltpu.make_async_copy(tab.at[:, pl.ds(0, _TPB)],
                                      bufs.at[slot], semt[slot]).wait()

        for s in range(_NBUF):
            fire(s, s)

        def jw_body(jw, carry):
            offs = offv[pl.ds(jw * _JBLK, _L)]
            cnts = cntv[pl.ds(jw * _JBLK, _L)]
            for jl in range(_JBLK):
                jloc = jw * _JBLK + jl
                slot = jl % _NBUF
                svec = jnp.full((_L,), slot, jnp.int32)
                wait_slot(jloc, slot)
                o0 = offs[jl]
                n = cnts[jl]

                def grp_body(g, c2, o0=o0, n=n, svec=svec):
                    acc2, f0, f1 = c2
                    fs = [f0, f1]
                    for k in range(_KG):
                        h = g * _KG + k
                        vmaski = ((h - n) >> 31) & 1
                        hs = h * vmaski + (n - 1) * (1 - vmaski)
                        rv = recv[pl.ds(o0 + hs, _L)]
                        rec_s = rv[0]
                        b = rec_s >> 7
                        u = rec_s & (_TPB - 1)
                        usplat = jnp.full((_L,), 0, jnp.int32) + u
                        vmaskf = vmaski.astype(jnp.float32)
                        vs = []
                        for c in range(D // _L):
                            v = plsc.load_gather(
                                bufs, [svec, dvecs[c], usplat])
                            vs.append(v)
                            acc2 = acc2 + (v * v) * vmaskf

                        @pl.when(vmaski > 0)
                        def _(k=k, b=b, vs=vs, fk=fs[k]):
                            @pl.when(fk >= 1)
                            def _():
                                pltpu.make_async_copy(
                                    out_flat.at[pl.ds(0, D)],
                                    colstage.at[k], semo[k]).wait()
                            for c in range(D // _L):
                                colstage[k, pl.ds(c * _L, _L)] = vs[c]
                            dst = pl.multiple_of(b * D, 8)
                            pltpu.async_copy(colstage.at[k],
                                             out_flat.at[pl.ds(dst, D)],
                                             semo[k])

                        fs[k] = fs[k] + vmaski
                    return (acc2, fs[0], fs[1])

                ng = (n + (_KG - 1)) >> 1
                carry = lax.fori_loop(0, ng, grp_body, carry)
                fire(jloc + _NBUF, slot)
            return carry

        carry = (jnp.zeros((_L,), jnp.float32), jnp.int32(0), jnp.int32(0))
        acc, f0, f1 = lax.fori_loop(0, _JW // _JBLK, jw_body, carry)

        for k, fk in enumerate((f0, f1)):
            @pl.when(fk >= 1)
            def _(k=k):
                pltpu.make_async_copy(out_flat.at[pl.ds(0, D)],
                                      colstage.at[k], semo[k]).wait()

        accst[...] = acc
        pltpu.sync_copy(accst, part_out.at[wid])

    return mf


def _prep(idx, J):
    n = idx.shape[0]
    j = idx >> 7
    skey = (j << 16) | jnp.arange(n, dtype=jnp.int32)
    s = jnp.sort(skey)
    order = s & ((1 << 16) - 1)
    rec = (order << 7) | (jnp.take(idx, order) & (_TPB - 1))
    cnt = jnp.bincount(j, length=J).astype(jnp.int32)
    off = (jnp.cumsum(cnt) - cnt).astype(jnp.int32)
    return rec.astype(jnp.int32), off, cnt


def kernel(user_list, pos_items, neg_items, user_table, item_table):
    B = user_list.shape[0]
    D = user_table.shape[1]
    NV = user_table.shape[0]
    info = plsc.get_sparse_core_info()
    J = info.num_cores * info.num_subcores * _JW

    scan_u = _build_scan(B, D, NV)
    scan_i = _build_scan(2 * B, D, item_table.shape[0])

    rec_u, off_u, cnt_u = _prep(user_list, J)
    out_u, parts_u = scan_u(user_table.T, rec_u, off_u, cnt_u)

    idx_i = jnp.concatenate([pos_items, neg_items])
    rec_i, off_i, cnt_i = _prep(idx_i, J)
    out_i, parts_i = scan_i(item_table.T, rec_i, off_i, cnt_i)

    user_emb = out_u.reshape(B, D)
    pn = out_i.reshape(2 * B, D)
    reg = (jnp.sum(parts_u) + jnp.sum(parts_i)) / B
    return (user_emb, pn[:B], pn[B:], reg)
